# Initial kernel scaffold; baseline (speedup 1.0000x reference)
import jax
import jax.numpy as jnp
from jax.experimental import pallas as pl


def kernel(feats, edge_index, W1, b1, W2, b2):
    raise NotImplementedError("write your pallas kernel here")


# speedup vs baseline: 5.1055x; 5.1055x over previous
import functools

import jax
import jax.numpy as jnp
from jax import lax
from jax.experimental import pallas as pl
from jax.experimental.pallas import tpu as pltpu
from jax.experimental.pallas import tpu_sc as plsc

_N = 10000
_NP = 10240
_E = 160000
_D1 = 256
_D2 = 128

_NC = 2
_NS = 16
_NW = _NC * _NS
_CH = 128
_NCH = _E // _CH
_CPC = _NCH // _NC
_TPS = (_CPC + _NS - 1) // _NS
_RPT = _NP // _NS

_RB3 = 10
_RB = _RB3 * 128
_GRID = _NP // _RB

_mesh = plsc.VectorSubcoreMesh(core_axis_name="c", subcore_axis_name="s")


def _deg_body(src_hbm, dst_hbm, hout_hbm, hin_hbm, idx_v, ho_v, hi_v):
    c = lax.axis_index("c")
    s = lax.axis_index("s")
    wid = s * _NC + c
    z16 = jnp.zeros((16,), jnp.float32)

    def zero(i, _):
        ho_v[pl.ds(i * 16, 16)] = z16
        hi_v[pl.ds(i * 16, 16)] = z16
        return 0

    lax.fori_loop(0, _NP // 16, zero, 0)

    ones16 = jnp.ones((16,), jnp.float32)

    def body(t, _):
        g = wid + t * _NW

        @pl.when(g < _NCH)
        def _():
            pltpu.sync_copy(src_hbm.at[pl.ds(g * _CH, _CH)], idx_v)
            for j in range(_CH // 16):
                plsc.addupdate_scatter(ho_v, [idx_v[pl.ds(j * 16, 16)]], ones16)
            pltpu.sync_copy(dst_hbm.at[pl.ds(g * _CH, _CH)], idx_v)
            for j in range(_CH // 16):
                plsc.addupdate_scatter(hi_v, [idx_v[pl.ds(j * 16, 16)]], ones16)

        return 0

    lax.fori_loop(0, (_NCH + _NW - 1) // _NW, body, 0)
    pltpu.sync_copy(ho_v, hout_hbm.at[wid])
    pltpu.sync_copy(hi_v, hin_hbm.at[wid])


_deg = pl.kernel(
    _deg_body,
    out_type=(
        jax.ShapeDtypeStruct((_NW, _NP), jnp.float32),
        jax.ShapeDtypeStruct((_NW, _NP), jnp.float32),
    ),
    mesh=_mesh,
    compiler_params=pltpu.CompilerParams(needs_layout_passes=False),
    scratch_types=[
        pltpu.VMEM((_CH,), jnp.int32),
        pltpu.VMEM((_NP,), jnp.float32),
        pltpu.VMEM((_NP,), jnp.float32),
    ],
)


def _agg_body(x_hbm, src_hbm, dst_hbm, out_hbm, idx_s, idx_d, rows_v, acc_sh, sem):
    c = lax.axis_index("c")
    s = lax.axis_index("s")
    z16 = jnp.zeros((16,), jnp.float32)

    def zero(i, _):
        for j in range(_D2 // 16):
            rows_v[i, pl.ds(j * 16, 16)] = z16
        return 0

    lax.fori_loop(0, _CH, zero, 0)

    base = s * _RPT
    for k in range(_RPT // _CH):
        pltpu.sync_copy(rows_v, acc_sh.at[pl.ds(base + k * _CH, _CH)])
    plsc.subcore_barrier()

    def body(t, _):
        j = s + t * _NS

        @pl.when(j < _CPC)
        def _():
            g = c * _CPC + j
            pltpu.sync_copy(src_hbm.at[pl.ds(g * _CH, _CH)], idx_s)
            pltpu.sync_copy(dst_hbm.at[pl.ds(g * _CH, _CH)], idx_d)
            pltpu.async_copy(x_hbm.at[idx_s], rows_v, sem).wait()
            pltpu.sync_copy(rows_v, acc_sh.at[idx_d], add=True)

        return 0

    lax.fori_loop(0, _TPS, body, 0)
    plsc.subcore_barrier()
    pltpu.sync_copy(acc_sh.at[pl.ds(base, _RPT)], out_hbm.at[c, pl.ds(base, _RPT)])


_agg = pl.kernel(
    _agg_body,
    out_type=jax.ShapeDtypeStruct((_NC, _NP, _D2), jnp.float32),
    mesh=_mesh,
    scratch_types=[
        pltpu.VMEM((_CH,), jnp.int32),
        pltpu.VMEM((_CH,), jnp.int32),
        pltpu.VMEM((_CH, _D2), jnp.float32),
        pltpu.VMEM_SHARED((_NP, _D2), jnp.float32),
        pltpu.SemaphoreType.DMA,
    ],
)


def _mm1_body(hout_ref, hin_ref, x_ref, w_ref, y_ref, so_ref, si_ref):
    do = jnp.sum(hout_ref[...], axis=1)
    di = jnp.sum(hin_ref[...], axis=1)
    so = lax.rsqrt(jnp.maximum(do, 1.0))
    si = lax.rsqrt(jnp.maximum(di, 1.0))
    so_ref[...] = so[None]
    si_ref[...] = si[None]
    x = x_ref[...] * so[:, :, None]
    y_ref[...] = jnp.dot(
        x.reshape(_RB, _D1), w_ref[...], preferred_element_type=jnp.float32
    )


def _mm1(hout3, hin3, feats3, w1):
    return pl.pallas_call(
        _mm1_body,
        grid=(_GRID,),
        in_specs=[
            pl.BlockSpec((_RB3, _NW, 128), lambda b: (b, 0, 0)),
            pl.BlockSpec((_RB3, _NW, 128), lambda b: (b, 0, 0)),
            pl.BlockSpec((_RB3, 128, _D1), lambda b: (b, 0, 0)),
            pl.BlockSpec((_D1, _D2), lambda b: (0, 0)),
        ],
        out_specs=[
            pl.BlockSpec((_RB, _D2), lambda b: (b, 0)),
            pl.BlockSpec((1, _RB3, 128), lambda b: (b, 0, 0)),
            pl.BlockSpec((1, _RB3, 128), lambda b: (b, 0, 0)),
        ],
        out_shape=[
            jax.ShapeDtypeStruct((_NP, _D2), jnp.float32),
            jax.ShapeDtypeStruct((_GRID, _RB3, 128), jnp.float32),
            jax.ShapeDtypeStruct((_GRID, _RB3, 128), jnp.float32),
        ],
    )(hout3, hin3, feats3, w1)


def _mm2_body(p0_ref, p1_ref, si_ref, so_ref, b1_ref, w_ref, y_ref):
    p = (p0_ref[...] + p1_ref[...]).reshape(_RB3, 128, _D2)
    si = si_ref[0]
    so = so_ref[0]
    b = b1_ref[...]
    h = jnp.maximum(p * si[:, :, None] + b[0][None, None, :], 0.0)
    h = h * so[:, :, None]
    y_ref[...] = jnp.dot(
        h.reshape(_RB, _D2), w_ref[...], preferred_element_type=jnp.float32
    )


def _mm2(p0, p1, si, so, b1_2d, w2):
    return pl.pallas_call(
        _mm2_body,
        grid=(_GRID,),
        in_specs=[
            pl.BlockSpec((_RB, _D2), lambda b: (b, 0)),
            pl.BlockSpec((_RB, _D2), lambda b: (b, 0)),
            pl.BlockSpec((1, _RB3, 128), lambda b: (b, 0, 0)),
            pl.BlockSpec((1, _RB3, 128), lambda b: (b, 0, 0)),
            pl.BlockSpec((1, _D2), lambda b: (0, 0)),
            pl.BlockSpec((_D2, _D2), lambda b: (0, 0)),
        ],
        out_specs=pl.BlockSpec((_RB, _D2), lambda b: (b, 0)),
        out_shape=jax.ShapeDtypeStruct((_NP, _D2), jnp.float32),
    )(p0, p1, si, so, b1_2d, w2)


def _out_body(q0_ref, q1_ref, si_ref, b2_ref, o_ref):
    q = (q0_ref[...] + q1_ref[...]).reshape(_RB3, 128, _D2)
    si = si_ref[0]
    b = b2_ref[...]
    o = jnp.maximum(q * si[:, :, None] + b[0][None, None, :], 0.0)
    o_ref[...] = o.reshape(_RB, _D2)


def _out(q0, q1, si, b2_2d):
    return pl.pallas_call(
        _out_body,
        grid=(_GRID,),
        in_specs=[
            pl.BlockSpec((_RB, _D2), lambda b: (b, 0)),
            pl.BlockSpec((_RB, _D2), lambda b: (b, 0)),
            pl.BlockSpec((1, _RB3, 128), lambda b: (b, 0, 0)),
            pl.BlockSpec((1, _D2), lambda b: (0, 0)),
        ],
        out_specs=pl.BlockSpec((_RB, _D2), lambda b: (b, 0)),
        out_shape=jax.ShapeDtypeStruct((_NP, _D2), jnp.float32),
    )(q0, q1, si, b2_2d)


def kernel(feats, edge_index, W1, b1, W2, b2):
    src = edge_index[0].astype(jnp.int32)
    dst = edge_index[1].astype(jnp.int32)
    feats3 = jnp.pad(feats, ((0, _NP - _N), (0, 0))).reshape(_NP // 128, 128, _D1)

    hout, hin = _deg(src, dst)
    hout3 = hout.reshape(_NW, _NP // 128, 128).transpose(1, 0, 2)
    hin3 = hin.reshape(_NW, _NP // 128, 128).transpose(1, 0, 2)

    x1, so, si = _mm1(hout3, hin3, feats3, W1)
    p = _agg(x1, src, dst)
    x2 = _mm2(p[0], p[1], si, so, b1.reshape(1, _D2), W2)
    q = _agg(x2, src, dst)
    out = _out(q[0], q[1], si, b2.reshape(1, _D2))
    return out[:_N]
